# Initial kernel scaffold; baseline (speedup 1.0000x reference)
import jax
import jax.numpy as jnp
from jax.experimental import pallas as pl


def kernel(x, table):
    raise NotImplementedError("write your pallas kernel here")


# speedup vs baseline: 41.2367x; 41.2367x over previous
import functools

import jax
import jax.numpy as jnp
from jax import lax
from jax.experimental import pallas as pl
from jax.experimental.pallas import tpu as pltpu
from jax.experimental.pallas import tpu_sc as plsc

_D = 32
_C = 20
_NC, _NS = 2, 16
_NW = _NC * _NS
_SZ = 128


def _sc_body(xt_hbm, table_hbm, out_hbm, idx_v, acc_v, sem):
    wid = lax.axis_index("s") * _NC + lax.axis_index("c")
    n_rows = out_hbm.shape[0]
    per_w = n_rows // _NW
    chunks = per_w // _SZ

    def chunk(i, carry):
        base = i * _SZ
        pltpu.sync_copy(xt_hbm.at[wid, :, pl.ds(base, _SZ)], idx_v)
        pltpu.async_copy(table_hbm.at[idx_v.at[0]], acc_v, sem).wait()
        cps = [
            pltpu.async_copy(table_hbm.at[idx_v.at[c]], acc_v, sem, add=True)
            for c in range(1, _C)
        ]
        for cp in cps:
            cp.wait()
        pltpu.sync_copy(acc_v, out_hbm.at[pl.ds(wid * per_w + base, _SZ)])
        return carry

    lax.fori_loop(0, chunks, chunk, 0)


def kernel(x, table):
    b, v, c = x.shape
    n = b * v
    xt = x.reshape(_NW, n // _NW, c).transpose(0, 2, 1)
    run = pl.kernel(
        _sc_body,
        out_type=jax.ShapeDtypeStruct((n, _D), jnp.float32),
        mesh=plsc.VectorSubcoreMesh(core_axis_name="c", subcore_axis_name="s"),
        scratch_types=[
            pltpu.VMEM((_C, _SZ), jnp.int32),
            pltpu.VMEM((_SZ, _D), jnp.float32),
            pltpu.SemaphoreType.DMA,
        ],
        compiler_params=pltpu.CompilerParams(use_tc_tiling_on_sc=False),
    )
    out = run(xt, table)
    return out.reshape(b, v, _D)


# speedup vs baseline: 47.2440x; 1.1457x over previous
import functools

import jax
import jax.numpy as jnp
from jax import lax
from jax.experimental import pallas as pl
from jax.experimental.pallas import tpu as pltpu
from jax.experimental.pallas import tpu_sc as plsc

_D = 32
_C = 20
_NC, _NS = 2, 16
_NW = _NC * _NS
_SZ = 1600


def _sc_body(xt_hbm, table_hbm, out_hbm, idx_v, acc_v, sem):
    wid = lax.axis_index("s") * _NC + lax.axis_index("c")
    n_rows = out_hbm.shape[0]
    per_w = n_rows // _NW
    chunks = per_w // _SZ

    def chunk(i, carry):
        base = i * _SZ
        pltpu.sync_copy(xt_hbm.at[wid, :, pl.ds(base, _SZ)], idx_v)
        pltpu.async_copy(table_hbm.at[idx_v.at[0]], acc_v, sem).wait()
        cps = [
            pltpu.async_copy(table_hbm.at[idx_v.at[c]], acc_v, sem, add=True)
            for c in range(1, _C)
        ]
        for cp in cps:
            cp.wait()
        pltpu.sync_copy(acc_v, out_hbm.at[pl.ds(wid * per_w + base, _SZ)])
        return carry

    lax.fori_loop(0, chunks, chunk, 0)


def kernel(x, table):
    b, v, c = x.shape
    n = b * v
    xt = x.reshape(_NW, n // _NW, c).transpose(0, 2, 1)
    run = pl.kernel(
        _sc_body,
        out_type=jax.ShapeDtypeStruct((n, _D), jnp.float32),
        mesh=plsc.VectorSubcoreMesh(core_axis_name="c", subcore_axis_name="s"),
        scratch_types=[
            pltpu.VMEM((_C, _SZ), jnp.int32),
            pltpu.VMEM((_SZ, _D), jnp.float32),
            pltpu.SemaphoreType.DMA,
        ],
        compiler_params=pltpu.CompilerParams(use_tc_tiling_on_sc=False),
    )
    out = run(xt, table)
    return out.reshape(b, v, _D)
